# baseline (speedup 1.0000x reference)
import functools

import jax
import jax.numpy as jnp
from jax import lax
from jax.experimental import pallas as pl
from jax.experimental.pallas import tpu as pltpu
from jax.experimental.pallas import tpu_sc as plsc

N = 10000
E = 160000
D = 256
H = 128
G = 128
RATIO = 0.8

NC = 2
NS = 16
NW = NC * NS

_MESH = plsc.VectorSubcoreMesh(core_axis_name="c", subcore_axis_name="s")


_ROWS_W = 320
_CH = 64
_NCH = _ROWS_W // _CH


@functools.partial(
    pl.kernel,
    mesh=_MESH,
    out_type=[
        jax.ShapeDtypeStruct((N, H), jnp.float32),
        jax.ShapeDtypeStruct((N, H), jnp.float32),
    ],
    scratch_types=[
        pltpu.VMEM((_ROWS_W,), jnp.int32),
        pltpu.VMEM((_CH, D), jnp.float32),
        pltpu.VMEM((_CH, H), jnp.float32),
        pltpu.VMEM((_CH, H), jnp.float32),
        pltpu.SemaphoreType.DMA,
    ],
)
def _emb_gather(idx_hbm, emb_hbm, xl_hbm, xr_hbm, idx_v, rows_v, hl_v, hr_v, sem):
    wid = lax.axis_index("s") * NC + lax.axis_index("c")
    start = jnp.minimum(wid * _ROWS_W, N - _ROWS_W)
    pltpu.sync_copy(idx_hbm.at[pl.ds(start, _ROWS_W)], idx_v)
    for c in range(_NCH):
        pltpu.async_copy(
            emb_hbm.at[idx_v.at[pl.ds(c * _CH, _CH)]], rows_v, sem
        ).wait()

        def _split(r, _):
            for k in range(H // 16):
                hl_v[r, pl.ds(k * 16, 16)] = rows_v[r, pl.ds(k * 16, 16)]
                hr_v[r, pl.ds(k * 16, 16)] = rows_v[r, pl.ds(H + k * 16, 16)]
            return _

        lax.fori_loop(0, _CH, _split, 0)
        pltpu.sync_copy(hl_v, xl_hbm.at[pl.ds(start + c * _CH, _CH)])
        pltpu.sync_copy(hr_v, xr_hbm.at[pl.ds(start + c * _CH, _CH)])


_EPT = E // NS
_EC = 128
_NFULL = _EPT // _EC
_NPAIR = _NFULL // 2
_ETAIL = _EPT - _NFULL * _EC
_RPT = 632


@functools.partial(
    pl.kernel,
    mesh=_MESH,
    out_type=[
        jax.ShapeDtypeStruct((N, H), jnp.float32),
        jax.ShapeDtypeStruct((N, H), jnp.float32),
    ],
    scratch_types=[
        pltpu.VMEM_SHARED((N, H), jnp.float32),
        pltpu.VMEM((_EC,), jnp.int32),
        pltpu.VMEM((_EC,), jnp.int32),
        pltpu.VMEM((_EC, H), jnp.float32),
        pltpu.VMEM((_EC,), jnp.int32),
        pltpu.VMEM((_EC,), jnp.int32),
        pltpu.VMEM((_EC, H), jnp.float32),
        pltpu.VMEM((_ETAIL,), jnp.int32),
        pltpu.VMEM((_ETAIL,), jnp.int32),
        pltpu.VMEM((_ETAIL, H), jnp.float32),
        pltpu.SemaphoreType.DMA,
        pltpu.SemaphoreType.DMA,
        pltpu.SemaphoreType.DMA,
        pltpu.SemaphoreType.DMA,
        pltpu.SemaphoreType.DMA,
        pltpu.SemaphoreType.DMA,
        pltpu.SemaphoreType.DMA,
    ],
)
def _conv_agg(xl_hbm, xr_hbm, src_hbm, dst_hbm, aggl_hbm, aggr_hbm,
              agg_sh, src0, dst0, rows0, src1, dst1, rows1,
              src_t, dst_t, rows_t,
              isem0, isem1, gsem0, gsem1, ssem0, ssem1, sem):
    cid = lax.axis_index("c")
    tid = lax.axis_index("s")

    def _zrow(r, carry):
        for k in range(H // 16):
            rows0[r, pl.ds(k * 16, 16)] = jnp.zeros((16,), jnp.float32)
        return carry

    lax.fori_loop(0, _EC, _zrow, 0)
    rbase = jnp.minimum(tid * _RPT, N - _RPT)
    for j in range(_RPT // _EC):
        pltpu.sync_copy(rows0, agg_sh.at[pl.ds(rbase + j * _EC, _EC)])
    pltpu.sync_copy(rows0.at[pl.ds(0, _RPT - (_RPT // _EC) * _EC)],
                    agg_sh.at[pl.ds(rbase + (_RPT // _EC) * _EC,
                                    _RPT - (_RPT // _EC) * _EC)])
    plsc.subcore_barrier()

    ebase = tid * _EPT
    xh = (xl_hbm, xr_hbm)

    def _issue_pair_idx(pi):
        c0 = ebase + (2 * pi) * _EC
        c1 = c0 + _EC
        pltpu.async_copy(src_hbm.at[pl.ds(c0, _EC)], src0, isem0)
        pltpu.async_copy(dst_hbm.at[pl.ds(c0, _EC)], dst0, isem0)
        pltpu.async_copy(src_hbm.at[pl.ds(c1, _EC)], src1, isem1)
        pltpu.async_copy(dst_hbm.at[pl.ds(c1, _EC)], dst1, isem1)

    def _wait_pair_idx():
        pltpu.make_async_copy(src_hbm.at[pl.ds(0, _EC)], src0, isem0).wait()
        pltpu.make_async_copy(dst_hbm.at[pl.ds(0, _EC)], dst0, isem0).wait()
        pltpu.make_async_copy(src_hbm.at[pl.ds(0, _EC)], src1, isem1).wait()
        pltpu.make_async_copy(dst_hbm.at[pl.ds(0, _EC)], dst1, isem1).wait()

    _issue_pair_idx(0)

    def _pair(pi, carry):
        _wait_pair_idx()

        @pl.when(cid == 0)
        def _():
            g0 = pltpu.async_copy(xl_hbm.at[src0], rows0, gsem0)
            g1 = pltpu.async_copy(xl_hbm.at[src1], rows1, gsem1)
            g0.wait()
            s0 = pltpu.async_copy(rows0, agg_sh.at[dst0], ssem0, add=True)
            g1.wait()
            s1 = pltpu.async_copy(rows1, agg_sh.at[dst1], ssem1, add=True)
            s0.wait(); s1.wait()

        @pl.when(cid == 1)
        def _():
            g0 = pltpu.async_copy(xr_hbm.at[src0], rows0, gsem0)
            g1 = pltpu.async_copy(xr_hbm.at[src1], rows1, gsem1)
            g0.wait()
            s0 = pltpu.async_copy(rows0, agg_sh.at[dst0], ssem0, add=True)
            g1.wait()
            s1 = pltpu.async_copy(rows1, agg_sh.at[dst1], ssem1, add=True)
            s0.wait(); s1.wait()

        @pl.when(pi < _NPAIR - 1)
        def _():
            _issue_pair_idx(pi + 1)

        return carry

    lax.fori_loop(0, _NPAIR, _pair, 0)

    toff = ebase + _NFULL * _EC
    pltpu.sync_copy(src_hbm.at[pl.ds(toff, _ETAIL)], src_t)
    pltpu.sync_copy(dst_hbm.at[pl.ds(toff, _ETAIL)], dst_t)

    @pl.when(cid == 0)
    def _():
        pltpu.async_copy(xl_hbm.at[src_t], rows_t, sem).wait()

    @pl.when(cid == 1)
    def _():
        pltpu.async_copy(xr_hbm.at[src_t], rows_t, sem).wait()

    pltpu.sync_copy(rows_t, agg_sh.at[dst_t], add=True)
    plsc.subcore_barrier()

    @pl.when(cid == 0)
    def _():
        pltpu.sync_copy(agg_sh.at[pl.ds(rbase, _RPT)],
                        aggl_hbm.at[pl.ds(rbase, _RPT)])

    @pl.when(cid == 1)
    def _():
        pltpu.sync_copy(agg_sh.at[pl.ds(rbase, _RPT)],
                        aggr_hbm.at[pl.ds(rbase, _RPT)])


def _mlp_body(r1, r2, r3, r4, k1, k2, k3, k4,
              w1_ref, b1_ref, w2_ref, b2_ref, w3_ref, b3_ref, out_ref):
    mxa = None
    mea = None
    for r_ref, k_ref in ((r1, k1), (r2, k2), (r3, k3), (r4, k4)):
        r = r_ref[...]
        kcol = k_ref[...][:, :1]
        mx = jnp.where(kcol > 0.0, r[:, :D], 0.0)
        me = r[:, D:] * (1.0 / jnp.maximum(kcol, 1.0))
        mxa = mx if mxa is None else mxa + mx
        mea = me if mea is None else mea + me
    w1t = w1_ref[...].T
    h1 = jnp.maximum(jnp.dot(mxa, w1t[:D], preferred_element_type=jnp.float32)
                     + jnp.dot(mea, w1t[D:], preferred_element_type=jnp.float32)
                     + b1_ref[...][None, :], 0.0)
    h2 = jnp.maximum(jnp.dot(h1, w2_ref[...].T, preferred_element_type=jnp.float32)
                     + b2_ref[...][None, :], 0.0)
    logit = jnp.sum(h2 * w3_ref[...], axis=1) + b3_ref[...][0]
    out_ref[...] = 1.0 / (1.0 + jnp.exp(-logit))


def _mlp(rs, ks, p):
    return pl.pallas_call(
        _mlp_body,
        out_shape=jax.ShapeDtypeStruct((G,), jnp.float32),
    )(rs[0], rs[1], rs[2], rs[3], ks[0], ks[1], ks[2], ks[3],
      p['l1w'], p['l1b'], p['l2w'], p['l2b'], p['l3w'], p['l3b'])


_RB = 1000


def _dense_body(aggl, aggr, xl, xr, wrel, wroot, b, pw,
                hl_o, hr_o, s_o, t_o):
    wr = wrel[...]
    wt = wroot[...]
    h = (jnp.dot(aggl[...], wr[:H], preferred_element_type=jnp.float32)
         + jnp.dot(aggr[...], wr[H:], preferred_element_type=jnp.float32)
         + jnp.dot(xl[...], wt[:H], preferred_element_type=jnp.float32)
         + jnp.dot(xr[...], wt[H:], preferred_element_type=jnp.float32)
         + b[...])
    h = jnp.maximum(h, 0.0)
    hl_o[...] = h[:, :H]
    hr_o[...] = h[:, H:]
    pwv = pw[...]
    nrm = jnp.sqrt(jnp.sum(pwv * pwv)) + 1e-16
    sc = jnp.dot(h, pwv.T, preferred_element_type=jnp.float32) / nrm
    s_o[...] = sc
    t_o[...] = jnp.tanh(sc)


def _conv_dense(aggl, aggr, xl, xr, wrel, wroot, b, pw):
    nb = N // _RB
    row = lambda i: (i, 0)
    fix = lambda i: (0, 0)
    return pl.pallas_call(
        _dense_body,
        grid=(nb,),
        in_specs=[
            pl.BlockSpec((_RB, H), row), pl.BlockSpec((_RB, H), row),
            pl.BlockSpec((_RB, H), row), pl.BlockSpec((_RB, H), row),
            pl.BlockSpec((D, D), fix), pl.BlockSpec((D, D), fix),
            pl.BlockSpec((1, D), fix), pl.BlockSpec((1, D), fix),
        ],
        out_specs=[
            pl.BlockSpec((_RB, H), row), pl.BlockSpec((_RB, H), row),
            pl.BlockSpec((_RB, 1), row), pl.BlockSpec((_RB, 1), row),
        ],
        out_shape=[
            jax.ShapeDtypeStruct((N, H), jnp.float32),
            jax.ShapeDtypeStruct((N, H), jnp.float32),
            jax.ShapeDtypeStruct((N, 1), jnp.float32),
            jax.ShapeDtypeStruct((N, 1), jnp.float32),
        ],
    )(aggl, aggr, xl, xr, wrel, wroot, b, pw)


P = 512
_GPW = G // NW


@functools.partial(
    pl.kernel,
    mesh=_MESH,
    out_type=[
        jax.ShapeDtypeStruct((G * P,), jnp.float32),
        jax.ShapeDtypeStruct((N + 16, H), jnp.float32),
        jax.ShapeDtypeStruct((N + 16, H), jnp.float32),
        jax.ShapeDtypeStruct((G * P, ), jnp.float32),
        jax.ShapeDtypeStruct((G * 16,), jnp.float32),
    ],
    scratch_types=[
        pltpu.VMEM((G + 16,), jnp.int32),
        pltpu.VMEM((G + 16,), jnp.int32),
        pltpu.VMEM((P + 8,), jnp.float32),
        pltpu.VMEM((P + 8,), jnp.float32),
        pltpu.VMEM((P,), jnp.float32),
        pltpu.VMEM((P,), jnp.float32),
        pltpu.VMEM((P,), jnp.float32),
        pltpu.VMEM((16, H), jnp.float32),
        pltpu.VMEM((16, H), jnp.float32),
        pltpu.VMEM((16, H), jnp.float32),
        pltpu.VMEM((16, H), jnp.float32),
        pltpu.VMEM((P,), jnp.float32),
        pltpu.VMEM((32,), jnp.int32),
        pltpu.SemaphoreType.DMA,
        pltpu.SemaphoreType.DMA,
    ],
)
def _topk_readout(score_hbm, tan_hbm, keep_hbm, off_hbm, tot_hbm,
                  hl_hbm, hr_hbm,
                  nk_hbm, xnl_hbm, xnr_hbm, r_hbm, k_hbm,
                  offv, totv, scoreb, tanb, keepb, nkb, mb,
                  hbl, hbr, xbl, xbr, rbuf, fold, sem, sem2):
    wid = lax.axis_index("s") * NC + lax.axis_index("c")

    def _lanesum(vec):
        fold[pl.ds(0, 16)] = vec
        a = fold[pl.ds(0, 16)] + fold[pl.ds(8, 16)]
        fold[pl.ds(0, 16)] = a
        a = fold[pl.ds(0, 16)] + fold[pl.ds(4, 16)]
        fold[pl.ds(0, 16)] = a
        a = fold[pl.ds(0, 16)] + fold[pl.ds(2, 16)]
        fold[pl.ds(0, 16)] = a
        a = fold[pl.ds(0, 16)] + fold[pl.ds(1, 16)]
        return a[0]

    pltpu.sync_copy(off_hbm, offv.at[pl.ds(0, G)])
    pltpu.sync_copy(tot_hbm, totv.at[pl.ds(0, G)])
    iota = lax.iota(jnp.int32, 16)
    zi = jnp.zeros((16,), jnp.int32)
    zf = jnp.zeros((16,), jnp.float32)
    for q in range(_GPW):
        g = wid * _GPW + q
        off = offv[pl.ds(g, 16)][0]
        n = totv[pl.ds(g, 16)][0]
        off_al = (off // 8) * 8
        sh = off - off_al
        d1 = pltpu.async_copy(score_hbm.at[pl.ds(off_al, P + 8)], scoreb, sem)
        d2 = pltpu.async_copy(tan_hbm.at[pl.ds(off_al, P + 8)], tanb, sem2)
        d3 = pltpu.async_copy(keep_hbm.at[pl.ds(g * P, P)], keepb, sem)
        d1.wait(); d2.wait(); d3.wait()
        nch = (n + 15) // 16

        def _cbody(ci, cvec):
            km = keepb[pl.ds(ci * 16, 16)] > 0.5
            valid = (ci * 16 + iota) < n
            return cvec + jnp.where(km & valid, 1, 0)

        cvec = lax.fori_loop(0, nch, _cbody, zi)
        cnt_f = _lanesum(cvec).astype(jnp.float32)
        t = jnp.float32(RATIO) * cnt_f
        ti = t.astype(jnp.int32)
        kint = ti + jnp.where(ti.astype(jnp.float32) < t, 1, 0)

        def _icbody(ic, carry):
            s_ch = scoreb[pl.ds(sh + ic * 16, 16)]
            kp_ch = keepb[pl.ds(ic * 16, 16)]
            t_ch = tanb[pl.ds(sh + ic * 16, 16)]
            nkv = zf
            mbv = zf
            for j in range(16):
                s_i = s_ch[j]
                kp_i = kp_ch[j]
                ig = ic * 16 + j

                def _rbody(ci, rvec):
                    sv = scoreb[pl.ds(sh + ci * 16, 16)]
                    km = keepb[pl.ds(ci * 16, 16)] > 0.5
                    lane = ci * 16 + iota
                    valid = lane < n
                    gt = sv > s_i
                    eq = (sv == s_i) & (lane < ig)
                    return rvec + jnp.where(km & valid & (gt | eq), 1, 0)

                rank = _lanesum(lax.fori_loop(0, nch, _rbody, zi))
                nk_i = jnp.where((kp_i > 0.5) & (rank < kint), 1.0, 0.0)
                nkv = jnp.where(iota == j, nk_i, nkv)
                mbv = jnp.where(iota == j, nk_i * t_ch[j], mbv)
            valid = (ic * 16 + iota) < n
            nkb[pl.ds(ic * 16, 16)] = jnp.where(valid, nkv, 0.0)
            mb[pl.ds(ic * 16, 16)] = jnp.where(valid, mbv, 0.0)
            return carry

        for kk in range(P // 16):
            nkb[pl.ds(kk * 16, 16)] = zf
            mb[pl.ds(kk * 16, 16)] = zf
        lax.fori_loop(0, nch, _icbody, 0)
        pltpu.sync_copy(nkb, nk_hbm.at[pl.ds(g * P, P)])

        neg = jnp.full((16,), -1e30, jnp.float32)
        init = (tuple(neg for _ in range(8)), tuple(neg for _ in range(8)),
                tuple(zf for _ in range(8)), tuple(zf for _ in range(8)))

        def _hbody(ci, carry):
            mxl, mxr, sml, smr = carry
            base = off + ci * 16
            idxg = jnp.minimum(base + iota, off + n - 1)
            ga = pltpu.async_copy(hl_hbm.at[idxg], hbl, sem)
            gb = pltpu.async_copy(hr_hbm.at[idxg], hbr, sem2)
            ga.wait(); gb.wait()
            mv = mb[pl.ds(ci * 16, 16)]
            nv = nkb[pl.ds(ci * 16, 16)]
            mxl = list(mxl); mxr = list(mxr); sml = list(sml); smr = list(smr)
            for j in range(16):
                m_j = mv[j]
                sel = nv[j] > 0.5
                for k in range(8):
                    v = hbl[j, pl.ds(k * 16, 16)] * m_j
                    xbl[j, pl.ds(k * 16, 16)] = v
                    mxl[k] = jnp.where(sel, jnp.maximum(mxl[k], v), mxl[k])
                    sml[k] = sml[k] + v
                    w = hbr[j, pl.ds(k * 16, 16)] * m_j
                    xbr[j, pl.ds(k * 16, 16)] = w
                    mxr[k] = jnp.where(sel, jnp.maximum(mxr[k], w), mxr[k])
                    smr[k] = smr[k] + w
            idxs = jnp.where(base + iota < off + n, base + iota, N)
            sa = pltpu.async_copy(xbl, xnl_hbm.at[idxs], sem)
            sb = pltpu.async_copy(xbr, xnr_hbm.at[idxs], sem2)
            sa.wait(); sb.wait()
            return (tuple(mxl), tuple(mxr), tuple(sml), tuple(smr))

        mxl, mxr, sml, smr = lax.fori_loop(0, nch, _hbody, init)
        for k in range(8):
            rbuf[pl.ds(k * 16, 16)] = mxl[k]
            rbuf[pl.ds(H + k * 16, 16)] = mxr[k]
            rbuf[pl.ds(2 * H + k * 16, 16)] = sml[k]
            rbuf[pl.ds(3 * H + k * 16, 16)] = smr[k]
        pltpu.sync_copy(rbuf, r_hbm.at[pl.ds(g * P, P)])
        rbuf[pl.ds(0, 16)] = zf + kint.astype(jnp.float32)
        pltpu.sync_copy(rbuf.at[pl.ds(0, 16)], k_hbm.at[pl.ds(g * 16, 16)])


def kernel(x, edge_index, batch, params):
    p = params
    src = edge_index[0].astype(jnp.int32)
    dst = edge_index[1].astype(jnp.int32)
    idx = x[:, 0].astype(jnp.int32)

    xl, xr = _emb_gather(idx, p['emb'])

    total = jnp.bincount(batch, length=G).astype(jnp.int32)
    offsets = (jnp.cumsum(total) - total).astype(jnp.int32)
    keep_flat = jnp.ones((G * P,), jnp.float32)
    pad = jnp.zeros((P + 16,), jnp.float32)
    rs = []
    ks = []
    for i in range(1, 5):
        aggl, aggr = _conv_agg(xl, xr, src, dst)
        hl, hr, s2, t2 = _conv_dense(
            aggl, aggr, xl, xr, p['c%d_wrel' % i], p['c%d_wroot' % i],
            p['c%d_b' % i].reshape(1, D), p['p%d_w' % i].reshape(1, D))
        score_f = jnp.concatenate([s2[:, 0], pad])
        tan_f = jnp.concatenate([t2[:, 0], pad])
        keep_flat, xl, xr, r, kc = _topk_readout(
            score_f, tan_f, keep_flat, offsets, total, hl, hr)
        rs.append(r.reshape(G, P))
        ks.append(kc.reshape(G, 16))
    return _mlp(rs, ks, p)


# speedup vs baseline: 1.0783x; 1.0783x over previous
import functools

import jax
import jax.numpy as jnp
from jax import lax
from jax.experimental import pallas as pl
from jax.experimental.pallas import tpu as pltpu
from jax.experimental.pallas import tpu_sc as plsc

N = 10000
E = 160000
D = 256
H = 128
G = 128
RATIO = 0.8

NC = 2
NS = 16
NW = NC * NS

_MESH = plsc.VectorSubcoreMesh(core_axis_name="c", subcore_axis_name="s")


_ROWS_W = 320
_CH = 64
_NCH = _ROWS_W // _CH


@functools.partial(
    pl.kernel,
    mesh=_MESH,
    out_type=[
        jax.ShapeDtypeStruct((N, H), jnp.float32),
        jax.ShapeDtypeStruct((N, H), jnp.float32),
    ],
    scratch_types=[
        pltpu.VMEM((_ROWS_W,), jnp.int32),
        pltpu.VMEM((_CH, D), jnp.float32),
        pltpu.VMEM((_CH, H), jnp.float32),
        pltpu.VMEM((_CH, H), jnp.float32),
        pltpu.SemaphoreType.DMA,
    ],
)
def _emb_gather(idx_hbm, emb_hbm, xl_hbm, xr_hbm, idx_v, rows_v, hl_v, hr_v, sem):
    wid = lax.axis_index("s") * NC + lax.axis_index("c")
    start = jnp.minimum(wid * _ROWS_W, N - _ROWS_W)
    pltpu.sync_copy(idx_hbm.at[pl.ds(start, _ROWS_W)], idx_v)
    for c in range(_NCH):
        pltpu.async_copy(
            emb_hbm.at[idx_v.at[pl.ds(c * _CH, _CH)]], rows_v, sem
        ).wait()

        def _split(r, _):
            for k in range(H // 16):
                hl_v[r, pl.ds(k * 16, 16)] = rows_v[r, pl.ds(k * 16, 16)]
                hr_v[r, pl.ds(k * 16, 16)] = rows_v[r, pl.ds(H + k * 16, 16)]
            return _

        lax.fori_loop(0, _CH, _split, 0)
        pltpu.sync_copy(hl_v, xl_hbm.at[pl.ds(start + c * _CH, _CH)])
        pltpu.sync_copy(hr_v, xr_hbm.at[pl.ds(start + c * _CH, _CH)])


_EPT = E // NS
_EC = 128
_NFULL = _EPT // _EC
_NPAIR = _NFULL // 2
_ETAIL = _EPT - _NFULL * _EC
_RPT = 632


@functools.partial(
    pl.kernel,
    mesh=_MESH,
    out_type=[
        jax.ShapeDtypeStruct((N, H), jnp.float32),
        jax.ShapeDtypeStruct((N, H), jnp.float32),
    ],
    scratch_types=[
        pltpu.VMEM_SHARED((N, H), jnp.float32),
        pltpu.VMEM((_EC,), jnp.int32),
        pltpu.VMEM((_EC,), jnp.int32),
        pltpu.VMEM((_EC, H), jnp.float32),
        pltpu.VMEM((_EC,), jnp.int32),
        pltpu.VMEM((_EC,), jnp.int32),
        pltpu.VMEM((_EC, H), jnp.float32),
        pltpu.VMEM((_EC,), jnp.int32),
        pltpu.VMEM((_EC,), jnp.int32),
        pltpu.VMEM((_ETAIL,), jnp.int32),
        pltpu.VMEM((_ETAIL,), jnp.int32),
        pltpu.VMEM((_ETAIL, H), jnp.float32),
        pltpu.SemaphoreType.DMA,
        pltpu.SemaphoreType.DMA,
        pltpu.SemaphoreType.DMA,
        pltpu.SemaphoreType.DMA,
        pltpu.SemaphoreType.DMA,
        pltpu.SemaphoreType.DMA,
        pltpu.SemaphoreType.DMA,
    ],
)
def _conv_agg(xl_hbm, xr_hbm, src_hbm, dst_hbm, aggl_hbm, aggr_hbm,
              agg_sh, src0, dst0, rows0, src1, dst1, rows1, dsts0, dsts1,
              src_t, dst_t, rows_t,
              isem0, isem1, gsem0, gsem1, ssem0, ssem1, sem):
    cid = lax.axis_index("c")
    tid = lax.axis_index("s")

    def _zrow(r, carry):
        for k in range(H // 16):
            rows0[r, pl.ds(k * 16, 16)] = jnp.zeros((16,), jnp.float32)
        return carry

    lax.fori_loop(0, _EC, _zrow, 0)
    rbase = jnp.minimum(tid * _RPT, N - _RPT)
    for j in range(_RPT // _EC):
        pltpu.sync_copy(rows0, agg_sh.at[pl.ds(rbase + j * _EC, _EC)])
    pltpu.sync_copy(rows0.at[pl.ds(0, _RPT - (_RPT // _EC) * _EC)],
                    agg_sh.at[pl.ds(rbase + (_RPT // _EC) * _EC,
                                    _RPT - (_RPT // _EC) * _EC)])
    plsc.subcore_barrier()

    ebase = tid * _EPT
    xh = (xl_hbm, xr_hbm)

    def _issue_pair_idx(pi):
        c0 = ebase + (2 * pi) * _EC
        c1 = c0 + _EC
        pltpu.async_copy(src_hbm.at[pl.ds(c0, _EC)], src0, isem0)
        pltpu.async_copy(dst_hbm.at[pl.ds(c0, _EC)], dst0, isem0)
        pltpu.async_copy(src_hbm.at[pl.ds(c1, _EC)], src1, isem1)
        pltpu.async_copy(dst_hbm.at[pl.ds(c1, _EC)], dst1, isem1)

    def _wait_pair_idx():
        pltpu.make_async_copy(src_hbm.at[pl.ds(0, _EC)], src0, isem0).wait()
        pltpu.make_async_copy(dst_hbm.at[pl.ds(0, _EC)], dst0, isem0).wait()
        pltpu.make_async_copy(src_hbm.at[pl.ds(0, _EC)], src1, isem1).wait()
        pltpu.make_async_copy(dst_hbm.at[pl.ds(0, _EC)], dst1, isem1).wait()

    _issue_pair_idx(0)

    def _drain_scatters():
        pltpu.make_async_copy(rows0, agg_sh.at[pl.ds(0, _EC)], ssem0).wait()
        pltpu.make_async_copy(rows1, agg_sh.at[pl.ds(0, _EC)], ssem1).wait()

    def _pair(pi, carry):
        _wait_pair_idx()

        @pl.when(pi > 0)
        def _():
            _drain_scatters()

        for k in range(_EC // 16):
            dsts0[pl.ds(k * 16, 16)] = dst0[pl.ds(k * 16, 16)]
            dsts1[pl.ds(k * 16, 16)] = dst1[pl.ds(k * 16, 16)]

        @pl.when(cid == 0)
        def _():
            g0 = pltpu.async_copy(xl_hbm.at[src0], rows0, gsem0)
            g1 = pltpu.async_copy(xl_hbm.at[src1], rows1, gsem1)
            g0.wait()
            pltpu.async_copy(rows0, agg_sh.at[dsts0], ssem0, add=True)
            g1.wait()
            pltpu.async_copy(rows1, agg_sh.at[dsts1], ssem1, add=True)

        @pl.when(cid == 1)
        def _():
            g0 = pltpu.async_copy(xr_hbm.at[src0], rows0, gsem0)
            g1 = pltpu.async_copy(xr_hbm.at[src1], rows1, gsem1)
            g0.wait()
            pltpu.async_copy(rows0, agg_sh.at[dsts0], ssem0, add=True)
            g1.wait()
            pltpu.async_copy(rows1, agg_sh.at[dsts1], ssem1, add=True)

        @pl.when(pi < _NPAIR - 1)
        def _():
            _issue_pair_idx(pi + 1)

        return carry

    lax.fori_loop(0, _NPAIR, _pair, 0)
    _drain_scatters()

    toff = ebase + _NFULL * _EC
    pltpu.sync_copy(src_hbm.at[pl.ds(toff, _ETAIL)], src_t)
    pltpu.sync_copy(dst_hbm.at[pl.ds(toff, _ETAIL)], dst_t)

    @pl.when(cid == 0)
    def _():
        pltpu.async_copy(xl_hbm.at[src_t], rows_t, sem).wait()

    @pl.when(cid == 1)
    def _():
        pltpu.async_copy(xr_hbm.at[src_t], rows_t, sem).wait()

    pltpu.sync_copy(rows_t, agg_sh.at[dst_t], add=True)
    plsc.subcore_barrier()

    @pl.when(cid == 0)
    def _():
        pltpu.sync_copy(agg_sh.at[pl.ds(rbase, _RPT)],
                        aggl_hbm.at[pl.ds(rbase, _RPT)])

    @pl.when(cid == 1)
    def _():
        pltpu.sync_copy(agg_sh.at[pl.ds(rbase, _RPT)],
                        aggr_hbm.at[pl.ds(rbase, _RPT)])


def _mlp_body(r1, r2, r3, r4, k1, k2, k3, k4,
              w1_ref, b1_ref, w2_ref, b2_ref, w3_ref, b3_ref, out_ref):
    mxa = None
    mea = None
    for r_ref, k_ref in ((r1, k1), (r2, k2), (r3, k3), (r4, k4)):
        r = r_ref[...]
        kcol = k_ref[...][:, :1]
        mx = jnp.where(kcol > 0.0, r[:, :D], 0.0)
        me = r[:, D:] * (1.0 / jnp.maximum(kcol, 1.0))
        mxa = mx if mxa is None else mxa + mx
        mea = me if mea is None else mea + me
    w1t = w1_ref[...].T
    h1 = jnp.maximum(jnp.dot(mxa, w1t[:D], preferred_element_type=jnp.float32)
                     + jnp.dot(mea, w1t[D:], preferred_element_type=jnp.float32)
                     + b1_ref[...][None, :], 0.0)
    h2 = jnp.maximum(jnp.dot(h1, w2_ref[...].T, preferred_element_type=jnp.float32)
                     + b2_ref[...][None, :], 0.0)
    logit = jnp.sum(h2 * w3_ref[...], axis=1) + b3_ref[...][0]
    out_ref[...] = 1.0 / (1.0 + jnp.exp(-logit))


def _mlp(rs, ks, p):
    return pl.pallas_call(
        _mlp_body,
        out_shape=jax.ShapeDtypeStruct((G,), jnp.float32),
    )(rs[0], rs[1], rs[2], rs[3], ks[0], ks[1], ks[2], ks[3],
      p['l1w'], p['l1b'], p['l2w'], p['l2b'], p['l3w'], p['l3b'])


_RB = 1000


def _dense_body(aggl, aggr, xl, xr, wrel, wroot, b, pw,
                hl_o, hr_o, s_o, t_o):
    wr = wrel[...]
    wt = wroot[...]
    h = (jnp.dot(aggl[...], wr[:H], preferred_element_type=jnp.float32)
         + jnp.dot(aggr[...], wr[H:], preferred_element_type=jnp.float32)
         + jnp.dot(xl[...], wt[:H], preferred_element_type=jnp.float32)
         + jnp.dot(xr[...], wt[H:], preferred_element_type=jnp.float32)
         + b[...])
    h = jnp.maximum(h, 0.0)
    hl_o[...] = h[:, :H]
    hr_o[...] = h[:, H:]
    pwv = pw[...]
    nrm = jnp.sqrt(jnp.sum(pwv * pwv)) + 1e-16
    sc = jnp.dot(h, pwv.T, preferred_element_type=jnp.float32) / nrm
    s_o[...] = sc
    t_o[...] = jnp.tanh(sc)


def _conv_dense(aggl, aggr, xl, xr, wrel, wroot, b, pw):
    nb = N // _RB
    row = lambda i: (i, 0)
    fix = lambda i: (0, 0)
    return pl.pallas_call(
        _dense_body,
        grid=(nb,),
        in_specs=[
            pl.BlockSpec((_RB, H), row), pl.BlockSpec((_RB, H), row),
            pl.BlockSpec((_RB, H), row), pl.BlockSpec((_RB, H), row),
            pl.BlockSpec((D, D), fix), pl.BlockSpec((D, D), fix),
            pl.BlockSpec((1, D), fix), pl.BlockSpec((1, D), fix),
        ],
        out_specs=[
            pl.BlockSpec((_RB, H), row), pl.BlockSpec((_RB, H), row),
            pl.BlockSpec((_RB, 1), row), pl.BlockSpec((_RB, 1), row),
        ],
        out_shape=[
            jax.ShapeDtypeStruct((N, H), jnp.float32),
            jax.ShapeDtypeStruct((N, H), jnp.float32),
            jax.ShapeDtypeStruct((N, 1), jnp.float32),
            jax.ShapeDtypeStruct((N, 1), jnp.float32),
        ],
    )(aggl, aggr, xl, xr, wrel, wroot, b, pw)


P = 512
_GPW = G // NW


@functools.partial(
    pl.kernel,
    mesh=_MESH,
    out_type=[
        jax.ShapeDtypeStruct((G * P,), jnp.float32),
        jax.ShapeDtypeStruct((N + 16, H), jnp.float32),
        jax.ShapeDtypeStruct((N + 16, H), jnp.float32),
        jax.ShapeDtypeStruct((G * P, ), jnp.float32),
        jax.ShapeDtypeStruct((G * 16,), jnp.float32),
    ],
    scratch_types=[
        pltpu.VMEM((G + 16,), jnp.int32),
        pltpu.VMEM((G + 16,), jnp.int32),
        pltpu.VMEM((P + 8,), jnp.float32),
        pltpu.VMEM((P + 8,), jnp.float32),
        pltpu.VMEM((P,), jnp.float32),
        pltpu.VMEM((P,), jnp.float32),
        pltpu.VMEM((P,), jnp.float32),
        pltpu.VMEM((16, H), jnp.float32),
        pltpu.VMEM((16, H), jnp.float32),
        pltpu.VMEM((16, H), jnp.float32),
        pltpu.VMEM((16, H), jnp.float32),
        pltpu.VMEM((P,), jnp.float32),
        pltpu.VMEM((32,), jnp.int32),
        pltpu.SemaphoreType.DMA,
        pltpu.SemaphoreType.DMA,
    ],
)
def _topk_readout(score_hbm, tan_hbm, keep_hbm, off_hbm, tot_hbm,
                  hl_hbm, hr_hbm,
                  nk_hbm, xnl_hbm, xnr_hbm, r_hbm, k_hbm,
                  offv, totv, scoreb, tanb, keepb, nkb, mb,
                  hbl, hbr, xbl, xbr, rbuf, fold, sem, sem2):
    wid = lax.axis_index("s") * NC + lax.axis_index("c")

    def _lanesum(vec):
        fold[pl.ds(0, 16)] = vec
        a = fold[pl.ds(0, 16)] + fold[pl.ds(8, 16)]
        fold[pl.ds(0, 16)] = a
        a = fold[pl.ds(0, 16)] + fold[pl.ds(4, 16)]
        fold[pl.ds(0, 16)] = a
        a = fold[pl.ds(0, 16)] + fold[pl.ds(2, 16)]
        fold[pl.ds(0, 16)] = a
        a = fold[pl.ds(0, 16)] + fold[pl.ds(1, 16)]
        return a[0]

    pltpu.sync_copy(off_hbm, offv.at[pl.ds(0, G)])
    pltpu.sync_copy(tot_hbm, totv.at[pl.ds(0, G)])
    iota = lax.iota(jnp.int32, 16)
    zi = jnp.zeros((16,), jnp.int32)
    zf = jnp.zeros((16,), jnp.float32)
    for q in range(_GPW):
        g = wid * _GPW + q
        off = offv[pl.ds(g, 16)][0]
        n = totv[pl.ds(g, 16)][0]
        off_al = (off // 8) * 8
        sh = off - off_al
        d1 = pltpu.async_copy(score_hbm.at[pl.ds(off_al, P + 8)], scoreb, sem)
        d2 = pltpu.async_copy(tan_hbm.at[pl.ds(off_al, P + 8)], tanb, sem2)
        d3 = pltpu.async_copy(keep_hbm.at[pl.ds(g * P, P)], keepb, sem)
        d1.wait(); d2.wait(); d3.wait()
        nch = (n + 15) // 16

        def _cbody(ci, cvec):
            km = keepb[pl.ds(ci * 16, 16)] > 0.5
            valid = (ci * 16 + iota) < n
            return cvec + jnp.where(km & valid, 1, 0)

        cvec = lax.fori_loop(0, nch, _cbody, zi)
        cnt_f = _lanesum(cvec).astype(jnp.float32)
        t = jnp.float32(RATIO) * cnt_f
        ti = t.astype(jnp.int32)
        kint = ti + jnp.where(ti.astype(jnp.float32) < t, 1, 0)

        def _icbody(ic, carry):
            s_ch = scoreb[pl.ds(sh + ic * 16, 16)]
            kp_ch = keepb[pl.ds(ic * 16, 16)]
            t_ch = tanb[pl.ds(sh + ic * 16, 16)]
            nkv = zf
            mbv = zf
            for j in range(16):
                s_i = s_ch[j]
                kp_i = kp_ch[j]
                ig = ic * 16 + j

                def _rbody(ci, rvec):
                    sv = scoreb[pl.ds(sh + ci * 16, 16)]
                    km = keepb[pl.ds(ci * 16, 16)] > 0.5
                    lane = ci * 16 + iota
                    valid = lane < n
                    gt = sv > s_i
                    eq = (sv == s_i) & (lane < ig)
                    return rvec + jnp.where(km & valid & (gt | eq), 1, 0)

                rank = _lanesum(lax.fori_loop(0, nch, _rbody, zi))
                nk_i = jnp.where((kp_i > 0.5) & (rank < kint), 1.0, 0.0)
                nkv = jnp.where(iota == j, nk_i, nkv)
                mbv = jnp.where(iota == j, nk_i * t_ch[j], mbv)
            valid = (ic * 16 + iota) < n
            nkb[pl.ds(ic * 16, 16)] = jnp.where(valid, nkv, 0.0)
            mb[pl.ds(ic * 16, 16)] = jnp.where(valid, mbv, 0.0)
            return carry

        for kk in range(P // 16):
            nkb[pl.ds(kk * 16, 16)] = zf
            mb[pl.ds(kk * 16, 16)] = zf
        lax.fori_loop(0, nch, _icbody, 0)
        pltpu.sync_copy(nkb, nk_hbm.at[pl.ds(g * P, P)])

        neg = jnp.full((16,), -1e30, jnp.float32)
        init = (tuple(neg for _ in range(8)), tuple(neg for _ in range(8)),
                tuple(zf for _ in range(8)), tuple(zf for _ in range(8)))

        def _hbody(ci, carry):
            mxl, mxr, sml, smr = carry
            base = off + ci * 16
            idxg = jnp.minimum(base + iota, off + n - 1)
            ga = pltpu.async_copy(hl_hbm.at[idxg], hbl, sem)
            gb = pltpu.async_copy(hr_hbm.at[idxg], hbr, sem2)
            ga.wait(); gb.wait()
            mv = mb[pl.ds(ci * 16, 16)]
            nv = nkb[pl.ds(ci * 16, 16)]
            mxl = list(mxl); mxr = list(mxr); sml = list(sml); smr = list(smr)
            for j in range(16):
                m_j = mv[j]
                sel = nv[j] > 0.5
                for k in range(8):
                    v = hbl[j, pl.ds(k * 16, 16)] * m_j
                    xbl[j, pl.ds(k * 16, 16)] = v
                    mxl[k] = jnp.where(sel, jnp.maximum(mxl[k], v), mxl[k])
                    sml[k] = sml[k] + v
                    w = hbr[j, pl.ds(k * 16, 16)] * m_j
                    xbr[j, pl.ds(k * 16, 16)] = w
                    mxr[k] = jnp.where(sel, jnp.maximum(mxr[k], w), mxr[k])
                    smr[k] = smr[k] + w
            idxs = jnp.where(base + iota < off + n, base + iota, N)
            sa = pltpu.async_copy(xbl, xnl_hbm.at[idxs], sem)
            sb = pltpu.async_copy(xbr, xnr_hbm.at[idxs], sem2)
            sa.wait(); sb.wait()
            return (tuple(mxl), tuple(mxr), tuple(sml), tuple(smr))

        mxl, mxr, sml, smr = lax.fori_loop(0, nch, _hbody, init)
        for k in range(8):
            rbuf[pl.ds(k * 16, 16)] = mxl[k]
            rbuf[pl.ds(H + k * 16, 16)] = mxr[k]
            rbuf[pl.ds(2 * H + k * 16, 16)] = sml[k]
            rbuf[pl.ds(3 * H + k * 16, 16)] = smr[k]
        pltpu.sync_copy(rbuf, r_hbm.at[pl.ds(g * P, P)])
        rbuf[pl.ds(0, 16)] = zf + kint.astype(jnp.float32)
        pltpu.sync_copy(rbuf.at[pl.ds(0, 16)], k_hbm.at[pl.ds(g * 16, 16)])


def kernel(x, edge_index, batch, params):
    p = params
    src = edge_index[0].astype(jnp.int32)
    dst = edge_index[1].astype(jnp.int32)
    idx = x[:, 0].astype(jnp.int32)

    xl, xr = _emb_gather(idx, p['emb'])

    total = jnp.bincount(batch, length=G).astype(jnp.int32)
    offsets = (jnp.cumsum(total) - total).astype(jnp.int32)
    keep_flat = jnp.ones((G * P,), jnp.float32)
    pad = jnp.zeros((P + 16,), jnp.float32)
    rs = []
    ks = []
    for i in range(1, 5):
        aggl, aggr = _conv_agg(xl, xr, src, dst)
        hl, hr, s2, t2 = _conv_dense(
            aggl, aggr, xl, xr, p['c%d_wrel' % i], p['c%d_wroot' % i],
            p['c%d_b' % i].reshape(1, D), p['p%d_w' % i].reshape(1, D))
        score_f = jnp.concatenate([s2[:, 0], pad])
        tan_f = jnp.concatenate([t2[:, 0], pad])
        keep_flat, xl, xr, r, kc = _topk_readout(
            score_f, tan_f, keep_flat, offsets, total, hl, hr)
        rs.append(r.reshape(G, P))
        ks.append(kc.reshape(G, 16))
    return _mlp(rs, ks, p)


# speedup vs baseline: 1.0796x; 1.0012x over previous
import functools

import jax
import jax.numpy as jnp
from jax import lax
from jax.experimental import pallas as pl
from jax.experimental.pallas import tpu as pltpu
from jax.experimental.pallas import tpu_sc as plsc

N = 10000
E = 160000
D = 256
H = 128
G = 128
RATIO = 0.8

NC = 2
NS = 16
NW = NC * NS

_MESH = plsc.VectorSubcoreMesh(core_axis_name="c", subcore_axis_name="s")


_ROWS_W = 320
_CH = 64
_NCH = _ROWS_W // _CH


@functools.partial(
    pl.kernel,
    mesh=_MESH,
    out_type=[
        jax.ShapeDtypeStruct((N, H), jnp.float32),
        jax.ShapeDtypeStruct((N, H), jnp.float32),
    ],
    scratch_types=[
        pltpu.VMEM((_ROWS_W,), jnp.int32),
        pltpu.VMEM((_CH, D), jnp.float32),
        pltpu.VMEM((_CH, H), jnp.float32),
        pltpu.VMEM((_CH, H), jnp.float32),
        pltpu.SemaphoreType.DMA,
    ],
)
def _emb_gather(idx_hbm, emb_hbm, xl_hbm, xr_hbm, idx_v, rows_v, hl_v, hr_v, sem):
    wid = lax.axis_index("s") * NC + lax.axis_index("c")
    start = jnp.minimum(wid * _ROWS_W, N - _ROWS_W)
    pltpu.sync_copy(idx_hbm.at[pl.ds(start, _ROWS_W)], idx_v)
    for c in range(_NCH):
        pltpu.async_copy(
            emb_hbm.at[idx_v.at[pl.ds(c * _CH, _CH)]], rows_v, sem
        ).wait()

        def _split(r, _):
            for k in range(H // 16):
                hl_v[r, pl.ds(k * 16, 16)] = rows_v[r, pl.ds(k * 16, 16)]
                hr_v[r, pl.ds(k * 16, 16)] = rows_v[r, pl.ds(H + k * 16, 16)]
            return _

        lax.fori_loop(0, _CH, _split, 0)
        pltpu.sync_copy(hl_v, xl_hbm.at[pl.ds(start + c * _CH, _CH)])
        pltpu.sync_copy(hr_v, xr_hbm.at[pl.ds(start + c * _CH, _CH)])


_EPT = E // NS
_EC = 128
_NFULL = _EPT // _EC
_NPAIR = _NFULL // 2
_ETAIL = _EPT - _NFULL * _EC
_RPT = 632


@functools.partial(
    pl.kernel,
    mesh=_MESH,
    out_type=[
        jax.ShapeDtypeStruct((N, H), jnp.float32),
        jax.ShapeDtypeStruct((N, H), jnp.float32),
    ],
    scratch_types=[
        pltpu.VMEM_SHARED((N, H), jnp.float32),
        pltpu.VMEM((_EC,), jnp.int32),
        pltpu.VMEM((_EC,), jnp.int32),
        pltpu.VMEM((_EC, H), jnp.float32),
        pltpu.VMEM((_EC,), jnp.int32),
        pltpu.VMEM((_EC,), jnp.int32),
        pltpu.VMEM((_EC, H), jnp.float32),
        pltpu.VMEM((_EC,), jnp.int32),
        pltpu.VMEM((_EC,), jnp.int32),
        pltpu.VMEM((_ETAIL,), jnp.int32),
        pltpu.VMEM((_ETAIL,), jnp.int32),
        pltpu.VMEM((_ETAIL, H), jnp.float32),
        pltpu.SemaphoreType.DMA,
        pltpu.SemaphoreType.DMA,
        pltpu.SemaphoreType.DMA,
        pltpu.SemaphoreType.DMA,
        pltpu.SemaphoreType.DMA,
        pltpu.SemaphoreType.DMA,
        pltpu.SemaphoreType.DMA,
    ],
)
def _conv_agg(xl_hbm, xr_hbm, src_hbm, dst_hbm, aggl_hbm, aggr_hbm,
              agg_sh, src0, dst0, rows0, src1, dst1, rows1, dsts0, dsts1,
              src_t, dst_t, rows_t,
              isem0, isem1, gsem0, gsem1, ssem0, ssem1, sem):
    cid = lax.axis_index("c")
    tid = lax.axis_index("s")

    def _zrow(r, carry):
        for k in range(H // 16):
            rows0[r, pl.ds(k * 16, 16)] = jnp.zeros((16,), jnp.float32)
        return carry

    lax.fori_loop(0, _EC, _zrow, 0)
    rbase = jnp.minimum(tid * _RPT, N - _RPT)
    for j in range(_RPT // _EC):
        pltpu.sync_copy(rows0, agg_sh.at[pl.ds(rbase + j * _EC, _EC)])
    pltpu.sync_copy(rows0.at[pl.ds(0, _RPT - (_RPT // _EC) * _EC)],
                    agg_sh.at[pl.ds(rbase + (_RPT // _EC) * _EC,
                                    _RPT - (_RPT // _EC) * _EC)])
    plsc.subcore_barrier()

    ebase = tid * _EPT
    xh = (xl_hbm, xr_hbm)

    def _issue_pair_idx(pi):
        c0 = ebase + (2 * pi) * _EC
        c1 = c0 + _EC
        pltpu.async_copy(src_hbm.at[pl.ds(c0, _EC)], src0, isem0)
        pltpu.async_copy(dst_hbm.at[pl.ds(c0, _EC)], dst0, isem0)
        pltpu.async_copy(src_hbm.at[pl.ds(c1, _EC)], src1, isem1)
        pltpu.async_copy(dst_hbm.at[pl.ds(c1, _EC)], dst1, isem1)

    def _wait_pair_idx():
        pltpu.make_async_copy(src_hbm.at[pl.ds(0, _EC)], src0, isem0).wait()
        pltpu.make_async_copy(dst_hbm.at[pl.ds(0, _EC)], dst0, isem0).wait()
        pltpu.make_async_copy(src_hbm.at[pl.ds(0, _EC)], src1, isem1).wait()
        pltpu.make_async_copy(dst_hbm.at[pl.ds(0, _EC)], dst1, isem1).wait()

    _issue_pair_idx(0)

    def _drain_scatters():
        pltpu.make_async_copy(rows0, agg_sh.at[pl.ds(0, _EC)], ssem0).wait()
        pltpu.make_async_copy(rows1, agg_sh.at[pl.ds(0, _EC)], ssem1).wait()

    def _pair(pi, carry):
        _wait_pair_idx()

        @pl.when(pi > 0)
        def _():
            _drain_scatters()

        for k in range(_EC // 16):
            dsts0[pl.ds(k * 16, 16)] = dst0[pl.ds(k * 16, 16)]
            dsts1[pl.ds(k * 16, 16)] = dst1[pl.ds(k * 16, 16)]

        @pl.when(cid == 0)
        def _():
            g0 = pltpu.async_copy(xl_hbm.at[src0], rows0, gsem0)
            g1 = pltpu.async_copy(xl_hbm.at[src1], rows1, gsem1)
            g0.wait()
            pltpu.async_copy(rows0, agg_sh.at[dsts0], ssem0, add=True)
            g1.wait()
            pltpu.async_copy(rows1, agg_sh.at[dsts1], ssem1, add=True)

        @pl.when(cid == 1)
        def _():
            g0 = pltpu.async_copy(xr_hbm.at[src0], rows0, gsem0)
            g1 = pltpu.async_copy(xr_hbm.at[src1], rows1, gsem1)
            g0.wait()
            pltpu.async_copy(rows0, agg_sh.at[dsts0], ssem0, add=True)
            g1.wait()
            pltpu.async_copy(rows1, agg_sh.at[dsts1], ssem1, add=True)

        @pl.when(pi < _NPAIR - 1)
        def _():
            _issue_pair_idx(pi + 1)

        return carry

    lax.fori_loop(0, _NPAIR, _pair, 0)
    _drain_scatters()

    toff = ebase + _NFULL * _EC
    pltpu.sync_copy(src_hbm.at[pl.ds(toff, _ETAIL)], src_t)
    pltpu.sync_copy(dst_hbm.at[pl.ds(toff, _ETAIL)], dst_t)

    @pl.when(cid == 0)
    def _():
        pltpu.async_copy(xl_hbm.at[src_t], rows_t, sem).wait()

    @pl.when(cid == 1)
    def _():
        pltpu.async_copy(xr_hbm.at[src_t], rows_t, sem).wait()

    pltpu.sync_copy(rows_t, agg_sh.at[dst_t], add=True)
    plsc.subcore_barrier()

    @pl.when(cid == 0)
    def _():
        pltpu.sync_copy(agg_sh.at[pl.ds(rbase, _RPT)],
                        aggl_hbm.at[pl.ds(rbase, _RPT)])

    @pl.when(cid == 1)
    def _():
        pltpu.sync_copy(agg_sh.at[pl.ds(rbase, _RPT)],
                        aggr_hbm.at[pl.ds(rbase, _RPT)])


def _mlp_body(r1, r2, r3, r4, k1, k2, k3, k4,
              w1_ref, b1_ref, w2_ref, b2_ref, w3_ref, b3_ref, out_ref):
    mxa = None
    mea = None
    for r_ref, k_ref in ((r1, k1), (r2, k2), (r3, k3), (r4, k4)):
        r = r_ref[...]
        kcol = k_ref[...][:, :1]
        mx = jnp.where(kcol > 0.0, r[:, :D], 0.0)
        me = r[:, D:] * (1.0 / jnp.maximum(kcol, 1.0))
        mxa = mx if mxa is None else mxa + mx
        mea = me if mea is None else mea + me
    w1t = w1_ref[...].T
    h1 = jnp.maximum(jnp.dot(mxa, w1t[:D], preferred_element_type=jnp.float32)
                     + jnp.dot(mea, w1t[D:], preferred_element_type=jnp.float32)
                     + b1_ref[...][None, :], 0.0)
    h2 = jnp.maximum(jnp.dot(h1, w2_ref[...].T, preferred_element_type=jnp.float32)
                     + b2_ref[...][None, :], 0.0)
    logit = jnp.sum(h2 * w3_ref[...], axis=1) + b3_ref[...][0]
    out_ref[...] = 1.0 / (1.0 + jnp.exp(-logit))


def _mlp(rs, ks, p):
    return pl.pallas_call(
        _mlp_body,
        out_shape=jax.ShapeDtypeStruct((G,), jnp.float32),
    )(rs[0], rs[1], rs[2], rs[3], ks[0], ks[1], ks[2], ks[3],
      p['l1w'], p['l1b'], p['l2w'], p['l2b'], p['l3w'], p['l3b'])


_RB = 1000


def _dense_body(aggl, aggr, xl, xr, wrel, wroot, b, pw,
                hl_o, hr_o, s_o, t_o):
    wr = wrel[...]
    wt = wroot[...]
    h = (jnp.dot(aggl[...], wr[:H], preferred_element_type=jnp.float32)
         + jnp.dot(aggr[...], wr[H:], preferred_element_type=jnp.float32)
         + jnp.dot(xl[...], wt[:H], preferred_element_type=jnp.float32)
         + jnp.dot(xr[...], wt[H:], preferred_element_type=jnp.float32)
         + b[...])
    h = jnp.maximum(h, 0.0)
    hl_o[...] = h[:, :H]
    hr_o[...] = h[:, H:]
    pwv = pw[...]
    nrm = jnp.sqrt(jnp.sum(pwv * pwv)) + 1e-16
    sc = jnp.dot(h, pwv.T, preferred_element_type=jnp.float32) / nrm
    s_o[...] = sc
    t_o[...] = jnp.tanh(sc)


def _conv_dense(aggl, aggr, xl, xr, wrel, wroot, b, pw):
    nb = N // _RB
    row = lambda i: (i, 0)
    fix = lambda i: (0, 0)
    return pl.pallas_call(
        _dense_body,
        grid=(nb,),
        in_specs=[
            pl.BlockSpec((_RB, H), row), pl.BlockSpec((_RB, H), row),
            pl.BlockSpec((_RB, H), row), pl.BlockSpec((_RB, H), row),
            pl.BlockSpec((D, D), fix), pl.BlockSpec((D, D), fix),
            pl.BlockSpec((1, D), fix), pl.BlockSpec((1, D), fix),
        ],
        out_specs=[
            pl.BlockSpec((_RB, H), row), pl.BlockSpec((_RB, H), row),
            pl.BlockSpec((_RB, 1), row), pl.BlockSpec((_RB, 1), row),
        ],
        out_shape=[
            jax.ShapeDtypeStruct((N, H), jnp.float32),
            jax.ShapeDtypeStruct((N, H), jnp.float32),
            jax.ShapeDtypeStruct((N, 1), jnp.float32),
            jax.ShapeDtypeStruct((N, 1), jnp.float32),
        ],
    )(aggl, aggr, xl, xr, wrel, wroot, b, pw)


P = 512
_GPW = G // NW


def _make_topk(write_x):
  deco = functools.partial(
    pl.kernel,
    mesh=_MESH,
    out_type=[
        jax.ShapeDtypeStruct((G * P,), jnp.float32),
        jax.ShapeDtypeStruct((N + 16, H), jnp.float32),
        jax.ShapeDtypeStruct((N + 16, H), jnp.float32),
        jax.ShapeDtypeStruct((G * P, ), jnp.float32),
        jax.ShapeDtypeStruct((G * 16,), jnp.float32),
    ],
    scratch_types=[
        pltpu.VMEM((G + 16,), jnp.int32),
        pltpu.VMEM((G + 16,), jnp.int32),
        pltpu.VMEM((P + 8,), jnp.float32),
        pltpu.VMEM((P + 8,), jnp.float32),
        pltpu.VMEM((P,), jnp.float32),
        pltpu.VMEM((P,), jnp.float32),
        pltpu.VMEM((P,), jnp.float32),
        pltpu.VMEM((16, H), jnp.float32),
        pltpu.VMEM((16, H), jnp.float32),
        pltpu.VMEM((16, H), jnp.float32),
        pltpu.VMEM((16, H), jnp.float32),
        pltpu.VMEM((P,), jnp.float32),
        pltpu.VMEM((32,), jnp.int32),
        pltpu.SemaphoreType.DMA,
        pltpu.SemaphoreType.DMA,
    ],
  )

  def _topk_readout(score_hbm, tan_hbm, keep_hbm, off_hbm, tot_hbm,
                    hl_hbm, hr_hbm,
                    nk_hbm, xnl_hbm, xnr_hbm, r_hbm, k_hbm,
                    offv, totv, scoreb, tanb, keepb, nkb, mb,
                    hbl, hbr, xbl, xbr, rbuf, fold, sem, sem2):
      wid = lax.axis_index("s") * NC + lax.axis_index("c")

      def _lanesum(vec):
          fold[pl.ds(0, 16)] = vec
          a = fold[pl.ds(0, 16)] + fold[pl.ds(8, 16)]
          fold[pl.ds(0, 16)] = a
          a = fold[pl.ds(0, 16)] + fold[pl.ds(4, 16)]
          fold[pl.ds(0, 16)] = a
          a = fold[pl.ds(0, 16)] + fold[pl.ds(2, 16)]
          fold[pl.ds(0, 16)] = a
          a = fold[pl.ds(0, 16)] + fold[pl.ds(1, 16)]
          return a[0]

      pltpu.sync_copy(off_hbm, offv.at[pl.ds(0, G)])
      pltpu.sync_copy(tot_hbm, totv.at[pl.ds(0, G)])
      iota = lax.iota(jnp.int32, 16)
      zi = jnp.zeros((16,), jnp.int32)
      zf = jnp.zeros((16,), jnp.float32)
      for q in range(_GPW):
          g = wid * _GPW + q
          off = offv[pl.ds(g, 16)][0]
          n = totv[pl.ds(g, 16)][0]
          off_al = (off // 8) * 8
          sh = off - off_al
          d1 = pltpu.async_copy(score_hbm.at[pl.ds(off_al, P + 8)], scoreb, sem)
          d2 = pltpu.async_copy(tan_hbm.at[pl.ds(off_al, P + 8)], tanb, sem2)
          d3 = pltpu.async_copy(keep_hbm.at[pl.ds(g * P, P)], keepb, sem)
          d1.wait(); d2.wait(); d3.wait()
          nch = (n + 15) // 16

          def _cbody(ci, cvec):
              km = keepb[pl.ds(ci * 16, 16)] > 0.5
              valid = (ci * 16 + iota) < n
              return cvec + jnp.where(km & valid, 1, 0)

          cvec = lax.fori_loop(0, nch, _cbody, zi)
          cnt_f = _lanesum(cvec).astype(jnp.float32)
          t = jnp.float32(RATIO) * cnt_f
          ti = t.astype(jnp.int32)
          kint = ti + jnp.where(ti.astype(jnp.float32) < t, 1, 0)

          def _icbody(ic, carry):
              s_ch = scoreb[pl.ds(sh + ic * 16, 16)]
              kp_ch = keepb[pl.ds(ic * 16, 16)]
              t_ch = tanb[pl.ds(sh + ic * 16, 16)]
              nkv = zf
              mbv = zf
              for j in range(16):
                  s_i = s_ch[j]
                  kp_i = kp_ch[j]
                  ig = ic * 16 + j

                  def _rbody(ci, rvec):
                      sv = scoreb[pl.ds(sh + ci * 16, 16)]
                      km = keepb[pl.ds(ci * 16, 16)] > 0.5
                      lane = ci * 16 + iota
                      valid = lane < n
                      gt = sv > s_i
                      eq = (sv == s_i) & (lane < ig)
                      return rvec + jnp.where(km & valid & (gt | eq), 1, 0)

                  rank = _lanesum(lax.fori_loop(0, nch, _rbody, zi))
                  nk_i = jnp.where((kp_i > 0.5) & (rank < kint), 1.0, 0.0)
                  nkv = jnp.where(iota == j, nk_i, nkv)
                  mbv = jnp.where(iota == j, nk_i * t_ch[j], mbv)
              valid = (ic * 16 + iota) < n
              nkb[pl.ds(ic * 16, 16)] = jnp.where(valid, nkv, 0.0)
              mb[pl.ds(ic * 16, 16)] = jnp.where(valid, mbv, 0.0)
              return carry

          for kk in range(P // 16):
              nkb[pl.ds(kk * 16, 16)] = zf
              mb[pl.ds(kk * 16, 16)] = zf
          lax.fori_loop(0, nch, _icbody, 0)
          pltpu.sync_copy(nkb, nk_hbm.at[pl.ds(g * P, P)])

          neg = jnp.full((16,), -1e30, jnp.float32)
          init = (tuple(neg for _ in range(8)), tuple(neg for _ in range(8)),
                  tuple(zf for _ in range(8)), tuple(zf for _ in range(8)))

          def _hbody(ci, carry):
              mxl, mxr, sml, smr = carry
              base = off + ci * 16
              idxg = jnp.minimum(base + iota, off + n - 1)
              ga = pltpu.async_copy(hl_hbm.at[idxg], hbl, sem)
              gb = pltpu.async_copy(hr_hbm.at[idxg], hbr, sem2)
              ga.wait(); gb.wait()
              mv = mb[pl.ds(ci * 16, 16)]
              nv = nkb[pl.ds(ci * 16, 16)]
              mxl = list(mxl); mxr = list(mxr); sml = list(sml); smr = list(smr)
              for j in range(16):
                  m_j = mv[j]
                  sel = nv[j] > 0.5
                  for k in range(8):
                      v = hbl[j, pl.ds(k * 16, 16)] * m_j
                      xbl[j, pl.ds(k * 16, 16)] = v
                      mxl[k] = jnp.where(sel, jnp.maximum(mxl[k], v), mxl[k])
                      sml[k] = sml[k] + v
                      w = hbr[j, pl.ds(k * 16, 16)] * m_j
                      xbr[j, pl.ds(k * 16, 16)] = w
                      mxr[k] = jnp.where(sel, jnp.maximum(mxr[k], w), mxr[k])
                      smr[k] = smr[k] + w
              idxs = jnp.where(base + iota < off + n, base + iota, N)
              sa = pltpu.async_copy(xbl, xnl_hbm.at[idxs], sem)
              sb = pltpu.async_copy(xbr, xnr_hbm.at[idxs], sem2)
              sa.wait(); sb.wait()
              return (tuple(mxl), tuple(mxr), tuple(sml), tuple(smr))

          mxl, mxr, sml, smr = lax.fori_loop(0, nch, _hbody, init)
          for k in range(8):
              rbuf[pl.ds(k * 16, 16)] = mxl[k]
              rbuf[pl.ds(H + k * 16, 16)] = mxr[k]
              rbuf[pl.ds(2 * H + k * 16, 16)] = sml[k]
              rbuf[pl.ds(3 * H + k * 16, 16)] = smr[k]
          pltpu.sync_copy(rbuf, r_hbm.at[pl.ds(g * P, P)])
          rbuf[pl.ds(0, 16)] = zf + kint.astype(jnp.float32)
          pltpu.sync_copy(rbuf.at[pl.ds(0, 16)], k_hbm.at[pl.ds(g * 16, 16)])


  return deco(_topk_readout)


_topk_full = _make_topk(True)
_topk_last = _make_topk(False)


def kernel(x, edge_index, batch, params):
    p = params
    src = edge_index[0].astype(jnp.int32)
    dst = edge_index[1].astype(jnp.int32)
    idx = x[:, 0].astype(jnp.int32)

    xl, xr = _emb_gather(idx, p['emb'])

    total = jnp.bincount(batch, length=G).astype(jnp.int32)
    offsets = (jnp.cumsum(total) - total).astype(jnp.int32)
    keep_flat = jnp.ones((G * P,), jnp.float32)
    pad = jnp.zeros((P + 16,), jnp.float32)
    rs = []
    ks = []
    for i in range(1, 5):
        aggl, aggr = _conv_agg(xl, xr, src, dst)
        hl, hr, s2, t2 = _conv_dense(
            aggl, aggr, xl, xr, p['c%d_wrel' % i], p['c%d_wroot' % i],
            p['c%d_b' % i].reshape(1, D), p['p%d_w' % i].reshape(1, D))
        score_f = jnp.concatenate([s2[:, 0], pad])
        tan_f = jnp.concatenate([t2[:, 0], pad])
        tk = _topk_full if i < 4 else _topk_last
        keep_flat, xl, xr, r, kc = tk(
            score_f, tan_f, keep_flat, offsets, total, hl, hr)
        rs.append(r.reshape(G, P))
        ks.append(kc.reshape(G, 16))
    return _mlp(rs, ks, p)
